# baseline (speedup 1.0000x reference)
import jax
import jax.numpy as jnp
from jax import lax
from jax.experimental import pallas as pl
from jax.experimental.pallas import tpu as pltpu
from jax.experimental.pallas import tpu_sc as plsc

_B, _S, _D = 4, 4096, 1024
_NC, _NS = 2, 16
_NW = _NC * _NS
_SPW = _S // _NW
_CH = 16
_NCHUNK = _SPW // _CH
_CHW = _CH * _D
_NSTEP = _NCHUNK * _B
_XR = 5
_PR = 2


def _sc_body(x_hbm, pe_hbm, out_hbm, *refs):
    xbs = refs[0:_XR]
    pbs = refs[_XR:_XR + _PR]
    sxs = refs[_XR + _PR:2 * _XR + _PR]
    sps = refs[2 * _XR + _PR:2 * _XR + 2 * _PR]
    sos = refs[2 * _XR + 2 * _PR:3 * _XR + 2 * _PR]

    wid = lax.axis_index("s") * _NC + lax.axis_index("c")
    s_base = wid * _SPW

    def x_off(t):
        b, g = t % _B, t // _B
        return (b * _S + s_base + g * _CH) * _D

    def fill_x(t):
        pltpu.async_copy(x_hbm.at[pl.ds(x_off(t), _CHW)], xbs[t % _XR], sxs[t % _XR])

    def fill_pe(g):
        pltpu.async_copy(pe_hbm.at[pl.ds((s_base + g * _CH) * _D, _CHW)],
                         pbs[g % _PR], sps[g % _PR])

    def wait_in(ref, sem):
        pltpu.make_async_copy(x_hbm.at[pl.ds(0, _CHW)], ref, sem).wait()

    def wait_out(p):
        pltpu.make_async_copy(xbs[p], out_hbm.at[pl.ds(0, _CHW)], sos[p]).wait()

    fill_x(0)
    fill_x(1)
    fill_x(2)
    fill_pe(0)
    for t in range(_NSTEP):
        b, g, p = t % _B, t // _B, t % _XR
        if t + 3 < _NSTEP:
            if t >= 2:
                wait_out((t + 3) % _XR)
            fill_x(t + 3)
        if b == 3 and g + 1 < _NCHUNK:
            fill_pe(g + 1)
        wait_in(xbs[p], sxs[p])
        if b == 0:
            wait_in(pbs[g % _PR], sps[g % _PR])
        xr, pr = xbs[p], pbs[g % _PR]

        @plsc.parallel_loop(0, _CHW, step=16, unroll=8)
        def _add16(i, xr=xr, pr=pr):
            plsc.addupdate(xr.at[pl.ds(i, 16)], pr[pl.ds(i, 16)])

        pltpu.async_copy(xbs[p], out_hbm.at[pl.ds(x_off(t), _CHW)], sos[p])
    for t in range(_NSTEP - _XR, _NSTEP):
        wait_out(t % _XR)


def kernel(x, pos_embedding):
    B, S, D = x.shape
    x1 = x.reshape(-1)
    pe1 = pos_embedding.reshape(-1)
    mesh = plsc.VectorSubcoreMesh(core_axis_name="c", subcore_axis_name="s")
    out = pl.kernel(
        _sc_body,
        out_type=jax.ShapeDtypeStruct((B * S * D,), x.dtype),
        mesh=mesh,
        scratch_types=(
            [pltpu.VMEM((_CHW,), jnp.float32)] * (_XR + _PR)
            + [pltpu.SemaphoreType.DMA] * (2 * _XR + 2 * _PR)
        ),
    )(x1, pe1)
    return out.reshape(B, S, D)


# speedup vs baseline: 1.8316x; 1.8316x over previous
import jax
import jax.numpy as jnp
from jax import lax
from jax.experimental import pallas as pl
from jax.experimental.pallas import tpu as pltpu
from jax.experimental.pallas import tpu_sc as plsc

_B, _S, _D = 4, 4096, 1024
_NC, _NS = 2, 16
_BLK = 128
_TR = _BLK // _NS
_NK = _S // _BLK
_NR = _NK * 2
_SR = 6


def _sc_body(x_hbm, pe_hbm, out_hbm, *refs):
    sps = refs[0:_SR]
    pbs = refs[_SR:_SR + 2]
    tbs = refs[_SR + 2:_SR + 4]
    sfs = refs[_SR + 4:2 * _SR + 4]
    spes = refs[2 * _SR + 4:2 * _SR + 6]
    sxis = refs[2 * _SR + 6:2 * _SR + 8]
    sds = refs[2 * _SR + 8:3 * _SR + 8]

    c = lax.axis_index("c")
    tid = lax.axis_index("s")

    def row0(t):
        k, b = t // 2, t % 2
        return (c * 2 + b) * _S + k * _BLK + tid * _TR

    def myslice(i):
        return sps[i].at[pl.ds(tid * _TR, _TR)]

    def fill(t):
        pltpu.async_copy(x_hbm.at[pl.ds(row0(t), _TR)], myslice(t % _SR),
                         sfs[t % _SR])

    def fill_pe(k):
        pltpu.async_copy(pe_hbm.at[pl.ds(k * _BLK + tid * _TR, _TR)],
                         pbs[k % 2], spes[k % 2])

    def pull(t):
        pltpu.async_copy(myslice(t % _SR), tbs[t % 2], sxis[t % 2])

    def wait_in(ref, sem):
        pltpu.make_async_copy(x_hbm.at[pl.ds(0, _TR)], ref, sem).wait()

    def drain(t):
        pltpu.async_copy(myslice(t % _SR), out_hbm.at[pl.ds(row0(t), _TR)],
                         sds[t % _SR])

    def wait_drain(i):
        pltpu.make_async_copy(myslice(i), out_hbm.at[pl.ds(0, _TR)],
                              sds[i]).wait()

    fill(0)
    fill(1)
    fill_pe(0)
    wait_in(myslice(0), sfs[0])
    pull(0)
    for t in range(_NR):
        k, b, p = t // 2, t % 2, t % _SR
        if t + 2 < _NR:
            if t >= 4:
                wait_drain((t + 2) % _SR)
            fill(t + 2)
        if b == 0 and k + 1 < _NK:
            fill_pe(k + 1)
        if t + 1 < _NR:
            wait_in(myslice((t + 1) % _SR), sfs[(t + 1) % _SR])
            pull(t + 1)
        wait_in(tbs[t % 2], sxis[t % 2])
        if b == 0:
            wait_in(pbs[k % 2], spes[k % 2])
        tb, pb = tbs[t % 2], pbs[k % 2]

        def row_body(r, carry, tb=tb, pb=pb):
            @plsc.parallel_loop(0, _D, step=16, unroll=8)
            def _add16(i):
                plsc.addupdate(tb.at[r, pl.ds(i, 16)], pb[r, pl.ds(i, 16)])
            return carry

        lax.fori_loop(0, _TR, row_body, 0)
        pltpu.sync_copy(tb, myslice(p))
        drain(t)
    for i in range(_SR):
        wait_drain(i)


def kernel(x, pos_embedding):
    B, S, D = x.shape
    x2 = x.reshape(B * S, D)
    mesh = plsc.VectorSubcoreMesh(core_axis_name="c", subcore_axis_name="s")
    out = pl.kernel(
        _sc_body,
        out_type=jax.ShapeDtypeStruct((B * S, D), x.dtype),
        mesh=mesh,
        scratch_types=(
            [pltpu.VMEM_SHARED((_BLK, _D), jnp.float32)] * _SR
            + [pltpu.VMEM((_TR, _D), jnp.float32)] * 4
            + [pltpu.SemaphoreType.DMA] * (2 * _SR + 4)
        ),
    )(x2, pos_embedding)
    return out.reshape(B, S, D)


# speedup vs baseline: 2.4599x; 1.3431x over previous
import jax
import jax.numpy as jnp
from jax import lax
from jax.experimental import pallas as pl
from jax.experimental.pallas import tpu as pltpu
from jax.experimental.pallas import tpu_sc as plsc

_B, _S, _D = 4, 4096, 1024
_NC, _NS = 2, 16
_BLK = 128
_TR = _BLK // _NS
_NK = _S // _BLK
_NR = _NK * 2
_SR = 6
_TB = 3


def _sc_body(x_hbm, pe_hbm, out_hbm, *refs):
    sps = refs[0:_SR]
    pbs = refs[_SR:_SR + 2]
    tbs = refs[_SR + 2:_SR + 2 + _TB]
    base = _SR + 2 + _TB
    sfs = refs[base:base + _SR]
    spes = refs[base + _SR:base + _SR + 2]
    sxis = refs[base + _SR + 2:base + _SR + 2 + _TB]
    sots = refs[base + _SR + 2 + _TB:base + _SR + 2 + 2 * _TB]

    c = lax.axis_index("c")
    tid = lax.axis_index("s")

    def row0(t):
        k, b = t // 2, t % 2
        return (c * 2 + b) * _S + k * _BLK + tid * _TR

    def myslice(i):
        return sps[i].at[pl.ds(tid * _TR, _TR)]

    def fill(t):
        pltpu.async_copy(x_hbm.at[pl.ds(row0(t), _TR)], myslice(t % _SR),
                         sfs[t % _SR])

    def fill_pe(k):
        pltpu.async_copy(pe_hbm.at[pl.ds(k * _BLK + tid * _TR, _TR)],
                         pbs[k % 2], spes[k % 2])

    def pull(t):
        pltpu.async_copy(myslice(t % _SR), tbs[t % _TB], sxis[t % _TB])

    def wait_in(ref, sem):
        pltpu.make_async_copy(x_hbm.at[pl.ds(0, _TR)], ref, sem).wait()

    def wait_out(i):
        pltpu.make_async_copy(tbs[i], out_hbm.at[pl.ds(0, _TR)],
                              sots[i]).wait()

    fill(0)
    fill(1)
    fill(2)
    fill_pe(0)
    wait_in(myslice(0), sfs[0])
    pull(0)
    for t in range(_NR):
        k, b = t // 2, t % 2
        if t + 3 < _NR:
            fill(t + 3)
        if b == 0 and k + 1 < _NK:
            fill_pe(k + 1)
        if t + 1 < _NR:
            if t >= 2:
                wait_out((t + 1) % _TB)
            wait_in(myslice((t + 1) % _SR), sfs[(t + 1) % _SR])
            pull(t + 1)
        wait_in(tbs[t % _TB], sxis[t % _TB])
        if b == 0:
            wait_in(pbs[k % 2], spes[k % 2])
        tb, pb = tbs[t % _TB], pbs[k % 2]

        def row_body(r, carry, tb=tb, pb=pb):
            @plsc.parallel_loop(0, _D, step=16, unroll=8)
            def _add16(i):
                plsc.addupdate(tb.at[r, pl.ds(i, 16)], pb[r, pl.ds(i, 16)])
            return carry

        lax.fori_loop(0, _TR, row_body, 0)
        pltpu.async_copy(tb, out_hbm.at[pl.ds(row0(t), _TR)], sots[t % _TB])
    for i in range(_TB):
        wait_out(i)


def kernel(x, pos_embedding):
    B, S, D = x.shape
    x2 = x.reshape(B * S, D)
    mesh = plsc.VectorSubcoreMesh(core_axis_name="c", subcore_axis_name="s")
    out = pl.kernel(
        _sc_body,
        out_type=jax.ShapeDtypeStruct((B * S, D), x.dtype),
        mesh=mesh,
        scratch_types=(
            [pltpu.VMEM_SHARED((_BLK, _D), jnp.float32)] * _SR
            + [pltpu.VMEM((_TR, _D), jnp.float32)] * (2 + _TB)
            + [pltpu.SemaphoreType.DMA] * (_SR + 2 + 2 * _TB)
        ),
    )(x2, pos_embedding)
    return out.reshape(B, S, D)
